# Initial kernel scaffold; baseline (speedup 1.0000x reference)
import jax
import jax.numpy as jnp
from jax.experimental import pallas as pl


def kernel(feats, adj, W1, b1, W2, b2, Wout, bout):
    raise NotImplementedError("write your pallas kernel here")


# speedup vs baseline: 1.0080x; 1.0080x over previous
import functools

import jax
import jax.numpy as jnp
from jax.experimental import pallas as pl
from jax.experimental.pallas import tpu as pltpu

N = 10000
H = 128
BLK = 400


def _y1_kernel(feats_ref, w1_ref, y1_ref):
    y1_ref[...] = jnp.dot(feats_ref[...], w1_ref[...],
                          preferred_element_type=jnp.float32)


def _pass1_kernel(adj_ref, y1_ref, b1_ref, w2_ref, h1_ref, z_ref):
    acc = jnp.dot(adj_ref[...], y1_ref[...],
                  preferred_element_type=jnp.float32)
    h1 = jnp.maximum(acc + b1_ref[...], 0.0)
    h1_ref[...] = h1
    z_ref[...] = jnp.dot(h1, w2_ref[...], preferred_element_type=jnp.float32)


def _pass2_kernel(adj_ref, z_ref, h1_ref, b2_ref, wo1_ref, wo2_ref,
                  bout_ref, out_ref):
    h2 = jnp.maximum(
        jnp.dot(adj_ref[...], z_ref[...], preferred_element_type=jnp.float32)
        + b2_ref[...], 0.0)
    out_ref[...] = (
        jnp.dot(h2, wo2_ref[...], preferred_element_type=jnp.float32)
        + jnp.dot(h1_ref[...], wo1_ref[...],
                  preferred_element_type=jnp.float32)
        + bout_ref[...])


@functools.partial(jax.jit, static_argnames=())
def kernel(feats, adj, W1, b1, W2, b2, Wout, bout):
    n_blocks = N // BLK
    full = lambda *_: (0, 0)

    y1 = pl.pallas_call(
        _y1_kernel,
        grid=(1,),
        in_specs=[pl.BlockSpec((N, H), full), pl.BlockSpec((H, H), full)],
        out_specs=pl.BlockSpec((N, H), full),
        out_shape=jax.ShapeDtypeStruct((N, H), jnp.float32),
    )(feats, W1)

    b1_2d = b1.reshape(1, H)
    b2_2d = b2.reshape(1, H)
    bout_2d = bout.reshape(1, H)
    wo1 = Wout[:H]
    wo2 = Wout[H:]

    row_blk = pl.BlockSpec((BLK, N), lambda i: (i, 0))
    out_blk = pl.BlockSpec((BLK, H), lambda i: (i, 0))
    wide = pl.BlockSpec((N, H), full)
    small = pl.BlockSpec((H, H), full)
    bias = pl.BlockSpec((1, H), full)

    h1, z = pl.pallas_call(
        _pass1_kernel,
        grid=(n_blocks,),
        in_specs=[row_blk, wide, bias, small],
        out_specs=[out_blk, out_blk],
        out_shape=[jax.ShapeDtypeStruct((N, H), jnp.float32),
                   jax.ShapeDtypeStruct((N, H), jnp.float32)],
        compiler_params=pltpu.CompilerParams(
            dimension_semantics=("arbitrary",),
            vmem_limit_bytes=100 * 1024 * 1024,
        ),
    )(adj, y1, b1_2d, W2)

    out = pl.pallas_call(
        _pass2_kernel,
        grid=(n_blocks,),
        in_specs=[row_blk, wide, out_blk, bias, small, small, bias],
        out_specs=out_blk,
        out_shape=jax.ShapeDtypeStruct((N, H), jnp.float32),
        compiler_params=pltpu.CompilerParams(
            dimension_semantics=("arbitrary",),
            vmem_limit_bytes=100 * 1024 * 1024,
        ),
    )(adj, z, h1, b2_2d, wo1, wo2, bout_2d)

    return out
